# baseline (speedup 1.0000x reference)
import functools
import numpy as np
import jax
import jax.numpy as jnp
from jax import lax
from jax.experimental import pallas as pl
from jax.experimental.pallas import tpu as pltpu
from jax.experimental.pallas import tpu_sc as plsc

_N_HEADS = 12
_NUM_FINE = 64
_B = 4
_N = 512
_L = 16
_CHUNKS = _N // _L
_TILES_PER_BATCH = 8
_I_PER_TILE = _N // _TILES_PER_BATCH


def _logits_body(imp_flat_ref, coarse_flat_ref, wq_ref, wk_ref,
                 logits_ref, p_ref):
    BQ, D = imp_flat_ref.shape
    H = _N_HEADS
    dh = D // H
    Q = BQ // _B

    wq = wq_ref[...]
    wk = wk_ref[...]
    qf = jnp.dot(imp_flat_ref[...], wq, preferred_element_type=jnp.float32)
    kf = jnp.dot(coarse_flat_ref[...], wk, preferred_element_type=jnp.float32)

    hid = jax.lax.broadcasted_iota(jnp.int32, (H, D), 0)
    did = jax.lax.broadcasted_iota(jnp.int32, (H, D), 1)
    hmask = (did // dh == hid).astype(jnp.float32)

    for b in range(_B):
        q_b = qf[(b + 1) * Q - 1:(b + 1) * Q, :]
        k = kf[b * _N:(b + 1) * _N, :]
        qmat = jnp.broadcast_to(q_b, (H, D)) * hmask
        s = jax.lax.dot_general(qmat, k, (((1,), (1,)), ((), ())),
                                preferred_element_type=jnp.float32)
        s = s / jnp.sqrt(jnp.float32(dh))
        probs = jax.nn.softmax(s, axis=-1)
        logits = jnp.mean(probs, axis=0, keepdims=True)
        logits_ref[b:b + 1, :] = logits
        p_ref[b:b + 1, :] = jax.nn.softmax(logits, axis=-1)


def _sc_rank_dispatch(logits_hbm, p_hbm, bi_hbm, sc_hbm,
                      lv, pv, myr, allr, biv, psv, scv, shared_r):
    c = lax.axis_index("c")
    s = lax.axis_index("s")
    lb = s // 8
    b = c * 2 + lb
    sub = s % 8

    pltpu.sync_copy(logits_hbm.at[b], lv)
    pltpu.sync_copy(p_hbm.at[b], pv)

    iota = lax.broadcasted_iota(jnp.int32, (_L,), 0)
    perms = [(iota + r) & (_L - 1) for r in range(_L)]

    for ci in range(_I_PER_TILE // _L):
        i0 = sub * _I_PER_TILE + ci * _L
        li = lv[pl.ds(i0, _L)]
        iidx = iota + i0

        def jbody(cj, cnt):
            j0 = cj * _L
            acc = cnt
            for r in range(_L):
                jidx = perms[r] + j0
                ljr = plsc.load_gather(lv, [jidx])
                gt = ljr > li
                tie = (ljr == li) & (jidx < iidx)
                acc = acc + jnp.where(gt | tie, 1, 0).astype(jnp.int32)
            return acc

        cnt = lax.fori_loop(0, _CHUNKS, jbody, jnp.zeros((_L,), jnp.int32))
        myr[pl.ds(ci * _L, _L)] = cnt

    pltpu.sync_copy(myr, shared_r.at[lb, pl.ds(sub * _I_PER_TILE,
                                               _I_PER_TILE)])
    plsc.subcore_barrier()

    @pl.when(sub == 0)
    def _():
        pltpu.sync_copy(shared_r.at[lb], allr)
        for ci in range(_CHUNKS):
            rk = allr[pl.ds(ci * _L, _L)]
            iv = iota + ci * _L
            plsc.store_scatter(biv, [rk], iv)
            plsc.store_scatter(psv, [rk], pv[pl.ds(ci * _L, _L)])
        for ci in range(_CHUNKS):
            ps_chunk = psv[pl.ds(ci * _L, _L)]
            if ci < _NUM_FINE // _L:
                sc = (1.0 + ps_chunk) - ps_chunk
            else:
                cs = 1.0 - ps_chunk
                sc = (1.0 + cs) - cs
            scv[pl.ds(ci * _L, _L)] = sc
        pltpu.sync_copy(biv, bi_hbm.at[b])
        pltpu.sync_copy(scv, sc_hbm.at[b])


def _sc_call(logits, p):
    mesh = plsc.VectorSubcoreMesh(core_axis_name="c", subcore_axis_name="s")
    fn = functools.partial(
        pl.kernel,
        mesh=mesh,
        compiler_params=pltpu.CompilerParams(needs_layout_passes=False),
        out_type=(
            jax.ShapeDtypeStruct((_B, _N), jnp.int32),
            jax.ShapeDtypeStruct((_B, _N), jnp.float32),
        ),
        scratch_types=[
            pltpu.VMEM((_N,), jnp.float32),
            pltpu.VMEM((_N,), jnp.float32),
            pltpu.VMEM((_I_PER_TILE,), jnp.int32),
            pltpu.VMEM((_N,), jnp.int32),
            pltpu.VMEM((_N,), jnp.int32),
            pltpu.VMEM((_N,), jnp.float32),
            pltpu.VMEM((_N,), jnp.float32),
            pltpu.VMEM_SHARED((2, _N), jnp.int32),
        ],
    )(_sc_rank_dispatch)
    return fn(logits, p)


def kernel(important_token_states, importance_mask, coarse_token_states,
           coarse_token_mask, important_token_positions,
           coarse_token_positions, Wq, Wk):
    del importance_mask, coarse_token_mask
    del important_token_positions, coarse_token_positions
    B, Q, D = important_token_states.shape
    N = coarse_token_states.shape[1]

    logits, p = pl.pallas_call(
        _logits_body,
        out_shape=(
            jax.ShapeDtypeStruct((B, N), jnp.float32),
            jax.ShapeDtypeStruct((B, N), jnp.float32),
        ),
    )(important_token_states.reshape(B * Q, D),
      coarse_token_states.reshape(B * N, D), Wq, Wk)

    bi, sc = _sc_call(logits, p)

    fine_block_indices = bi[:, :_NUM_FINE]
    coarse_block_indices = bi[:, _NUM_FINE:]
    fine_block_scores = sc[:, :_NUM_FINE]
    coarse_block_scores = sc[:, _NUM_FINE:]
    return (fine_block_indices, coarse_block_indices, fine_block_scores,
            coarse_block_scores)


# speedup vs baseline: 1.0508x; 1.0508x over previous
import functools
import numpy as np
import jax
import jax.numpy as jnp
from jax import lax
from jax.experimental import pallas as pl
from jax.experimental.pallas import tpu as pltpu
from jax.experimental.pallas import tpu_sc as plsc

_N_HEADS = 12
_NUM_FINE = 64
_B = 4
_N = 512
_L = 16
_CHUNKS = _N // _L
_TILES_PER_BATCH = 8
_I_PER_TILE = _N // _TILES_PER_BATCH


def _logits_body(imp_flat_ref, coarse_flat_ref, wq_ref, wk_ref,
                 logits_ref, p_ref):
    BQ, D = imp_flat_ref.shape
    H = _N_HEADS
    dh = D // H
    Q = BQ // _B

    wq = wq_ref[...]
    wk = wk_ref[...]
    qf = jnp.dot(imp_flat_ref[...], wq, preferred_element_type=jnp.float32)
    kf = jnp.dot(coarse_flat_ref[...], wk, preferred_element_type=jnp.float32)

    hid = jax.lax.broadcasted_iota(jnp.int32, (H, D), 0)
    did = jax.lax.broadcasted_iota(jnp.int32, (H, D), 1)
    hmask = (did // dh == hid).astype(jnp.float32)

    for b in range(_B):
        q_b = qf[(b + 1) * Q - 1:(b + 1) * Q, :]
        k = kf[b * _N:(b + 1) * _N, :]
        qmat = jnp.broadcast_to(q_b, (H, D)) * hmask
        s = jax.lax.dot_general(qmat, k, (((1,), (1,)), ((), ())),
                                preferred_element_type=jnp.float32)
        s = s / jnp.sqrt(jnp.float32(dh))
        probs = jax.nn.softmax(s, axis=-1)
        logits = jnp.mean(probs, axis=0, keepdims=True)
        logits_ref[b:b + 1, :] = logits
        p_ref[b:b + 1, :] = jax.nn.softmax(logits, axis=-1)


def _sc_rank_dispatch(logits_hbm, p_hbm, bi_hbm, sc_hbm,
                      lv, pv, myr, allr, biv, psv, scv, shared_r):
    c = lax.axis_index("c")
    s = lax.axis_index("s")
    lb = s // 8
    b = c * 2 + lb
    sub = s % 8

    pltpu.sync_copy(logits_hbm.at[b], lv)
    pltpu.sync_copy(p_hbm.at[b], pv)

    iota = lax.broadcasted_iota(jnp.int32, (_L,), 0)
    perms = [(iota + r) & (_L - 1) for r in range(_L)]
    tmasks = [perms[r] < iota for r in range(_L)]
    one = jnp.ones((_L,), jnp.int32)
    zero = jnp.zeros((_L,), jnp.int32)

    for ci in range(_I_PER_TILE // _L):
        i0 = sub * _I_PER_TILE + ci * _L
        li = lv[pl.ds(i0, _L)]
        diag_cj = sub * (_I_PER_TILE // _L) + ci

        def jbody_ge(cj, cnt):
            j0 = cj * _L
            acc = cnt
            for r in range(_L):
                ljr = plsc.load_gather(lv, [perms[r] + j0])
                acc = acc + jnp.where(ljr >= li, one, zero)
            return acc

        def jbody_gt(cj, cnt):
            j0 = cj * _L
            acc = cnt
            for r in range(_L):
                ljr = plsc.load_gather(lv, [perms[r] + j0])
                acc = acc + jnp.where(ljr > li, one, zero)
            return acc

        cnt = lax.fori_loop(0, diag_cj, jbody_ge, zero)
        cnt = lax.fori_loop(diag_cj + 1, _CHUNKS, jbody_gt, cnt)
        for r in range(_L):
            ljr = plsc.load_gather(lv, [perms[r] + i0])
            cond = (ljr > li) | ((ljr == li) & tmasks[r])
            cnt = cnt + jnp.where(cond, one, zero)
        myr[pl.ds(ci * _L, _L)] = cnt

    pltpu.sync_copy(myr, shared_r.at[lb, pl.ds(sub * _I_PER_TILE,
                                               _I_PER_TILE)])
    plsc.subcore_barrier()

    @pl.when(sub == 0)
    def _():
        pltpu.sync_copy(shared_r.at[lb], allr)
        for ci in range(_CHUNKS):
            rk = allr[pl.ds(ci * _L, _L)]
            iv = iota + ci * _L
            plsc.store_scatter(biv, [rk], iv)
            plsc.store_scatter(psv, [rk], pv[pl.ds(ci * _L, _L)])
        for ci in range(_CHUNKS):
            ps_chunk = psv[pl.ds(ci * _L, _L)]
            if ci < _NUM_FINE // _L:
                sc = (1.0 + ps_chunk) - ps_chunk
            else:
                cs = 1.0 - ps_chunk
                sc = (1.0 + cs) - cs
            scv[pl.ds(ci * _L, _L)] = sc
        pltpu.sync_copy(biv, bi_hbm.at[b])
        pltpu.sync_copy(scv, sc_hbm.at[b])


def _sc_call(logits, p):
    mesh = plsc.VectorSubcoreMesh(core_axis_name="c", subcore_axis_name="s")
    fn = functools.partial(
        pl.kernel,
        mesh=mesh,
        compiler_params=pltpu.CompilerParams(needs_layout_passes=False),
        out_type=(
            jax.ShapeDtypeStruct((_B, _N), jnp.int32),
            jax.ShapeDtypeStruct((_B, _N), jnp.float32),
        ),
        scratch_types=[
            pltpu.VMEM((_N,), jnp.float32),
            pltpu.VMEM((_N,), jnp.float32),
            pltpu.VMEM((_I_PER_TILE,), jnp.int32),
            pltpu.VMEM((_N,), jnp.int32),
            pltpu.VMEM((_N,), jnp.int32),
            pltpu.VMEM((_N,), jnp.float32),
            pltpu.VMEM((_N,), jnp.float32),
            pltpu.VMEM_SHARED((2, _N), jnp.int32),
        ],
    )(_sc_rank_dispatch)
    return fn(logits, p)


def kernel(important_token_states, importance_mask, coarse_token_states,
           coarse_token_mask, important_token_positions,
           coarse_token_positions, Wq, Wk):
    del importance_mask, coarse_token_mask
    del important_token_positions, coarse_token_positions
    B, Q, D = important_token_states.shape
    N = coarse_token_states.shape[1]

    logits, p = pl.pallas_call(
        _logits_body,
        out_shape=(
            jax.ShapeDtypeStruct((B, N), jnp.float32),
            jax.ShapeDtypeStruct((B, N), jnp.float32),
        ),
    )(important_token_states.reshape(B * Q, D),
      coarse_token_states.reshape(B * N, D), Wq, Wk)

    bi, sc = _sc_call(logits, p)

    fine_block_indices = bi[:, :_NUM_FINE]
    coarse_block_indices = bi[:, _NUM_FINE:]
    fine_block_scores = sc[:, :_NUM_FINE]
    coarse_block_scores = sc[:, _NUM_FINE:]
    return (fine_block_indices, coarse_block_indices, fine_block_scores,
            coarse_block_scores)
